# Initial kernel scaffold; baseline (speedup 1.0000x reference)
import jax
import jax.numpy as jnp
from jax.experimental import pallas as pl


def kernel(x, eps, pmeans, psigmas, pvalues):
    raise NotImplementedError("write your pallas kernel here")


# speedup vs baseline: 2.0885x; 2.0885x over previous
import functools

import jax
import jax.numpy as jnp
from jax import lax
from jax.experimental import pallas as pl
from jax.experimental.pallas import tpu as pltpu
from jax.experimental.pallas import tpu_sc as plsc

N = 16384
B = 64
FLAT = 2 * N
SCALE = float(N - 1)
SIGMA_BOOST = 2.0
EPSILON = 1e-7

N_BLK = 2048
L = 16


def _tc_body(pm_ref, ps_ref, eps_ref, means_ref, sig_ref, samp_ref, idx_ref):
    m = jax.nn.sigmoid(pm_ref[...]) * SCALE
    s = jax.nn.softplus(ps_ref[...] + SIGMA_BOOST) + EPSILON
    eps = eps_ref[...]
    means_ref[...] = jnp.broadcast_to(m, eps.shape)
    sig_ref[...] = jnp.broadcast_to(s, eps.shape)
    samp = m + s * eps
    samp_ref[...] = samp
    idx = jnp.clip(jnp.round(samp), 0.0, SCALE).astype(jnp.int32)
    idx_ref[...] = idx.astype(jnp.int16)


_tc_call = pl.pallas_call(
    _tc_body,
    grid=(FLAT // N_BLK,),
    in_specs=[
        pl.BlockSpec((1, N_BLK), lambda i: (0, i)),
        pl.BlockSpec((1, N_BLK), lambda i: (0, i)),
        pl.BlockSpec((B, N_BLK), lambda i: (0, i)),
    ],
    out_specs=[
        pl.BlockSpec((B, N_BLK), lambda i: (0, i)),
        pl.BlockSpec((B, N_BLK), lambda i: (0, i)),
        pl.BlockSpec((B, N_BLK), lambda i: (0, i)),
        pl.BlockSpec((B, N_BLK), lambda i: (0, i)),
    ],
    out_shape=[
        jax.ShapeDtypeStruct((B, FLAT), jnp.float32),
        jax.ShapeDtypeStruct((B, FLAT), jnp.float32),
        jax.ShapeDtypeStruct((B, FLAT), jnp.float32),
        jax.ShapeDtypeStruct((B, FLAT), jnp.int16),
    ],
)


_NC = 2
_NS = 16
_ROWS_PER_W = B // (_NC * _NS)

@functools.cache
def _get_sc_contract():
    mesh = plsc.VectorSubcoreMesh(
        core_axis_name="c", subcore_axis_name="s",
        num_cores=_NC, num_subcores=_NS,
    )

    @functools.partial(
        pl.kernel,
        out_type=jax.ShapeDtypeStruct((B, N), jnp.float32),
        mesh=mesh,
        scratch_types=[
            pltpu.VMEM((N,), jnp.float32),
            pltpu.VMEM((N,), jnp.int32),
            pltpu.VMEM((N,), jnp.float32),
            pltpu.VMEM((N,), jnp.float32),
        ],
        compiler_params=pltpu.CompilerParams(needs_layout_passes=False),
    )
    def _sc_contract(x_hbm, idx_hbm, pv_hbm, y_hbm, xv, iv, yv, pv):
        wid = lax.axis_index("s") * _NC + lax.axis_index("c")
        pltpu.sync_copy(pv_hbm, pv)
        for rep in range(_ROWS_PER_W):
            b = wid * _ROWS_PER_W + rep
            pltpu.sync_copy(x_hbm.at[b], xv)
            pltpu.sync_copy(idx_hbm.at[pl.ds(b * N, N)], iv)

            zero = jnp.zeros((L,), jnp.float32)

            def zbody(i, carry):
                yv[pl.ds(i * L, L)] = zero
                return carry

            lax.fori_loop(0, N // L, zbody, 0)

            def body(k, carry):
                w = iv[pl.ds(k * L, L)]
                r = lax.bitwise_and(w, 0xFFFF)
                c = lax.shift_right_logical(w, 16)
                g = plsc.load_gather(xv, [c])
                v = pv[pl.ds(k * L, L)]
                plsc.addupdate_scatter(yv, [r], g * v)
                return carry

            lax.fori_loop(0, N // L, body, 0)
            pltpu.sync_copy(yv, y_hbm.at[b])

    return _sc_contract


def kernel(x, eps, pmeans, psigmas, pvalues):
    pm_flat = pmeans.reshape(1, FLAT)
    psig_rep = jnp.repeat(psigmas, 2).reshape(1, FLAT)
    eps_flat = eps.reshape(B, FLAT)
    means_f, sig_f, samp_f, idx16 = _tc_call(pm_flat, psig_rep, eps_flat)
    idxw = lax.bitcast_convert_type(idx16.reshape(B * N, 2), jnp.int32)
    y = _get_sc_contract()(x, idxw, pvalues)
    return (
        y,
        means_f.reshape(B, N, 2),
        sig_f.reshape(B, N, 2),
        samp_f.reshape(B, N, 2),
    )


# speedup vs baseline: 10.3645x; 4.9626x over previous
import functools

import jax
import jax.numpy as jnp
from jax import lax
from jax.experimental import pallas as pl
from jax.experimental.pallas import tpu as pltpu
from jax.experimental.pallas import tpu_sc as plsc

N = 16384
B = 64
FLAT = 2 * N
SCALE = float(N - 1)
SIGMA_BOOST = 2.0
EPSILON = 1e-7

N_BLK = 2048
L = 16


def _tc_body(pm_ref, ps_ref, eps_ref, means_ref, sig_ref, samp_ref, idx_ref):
    m = jax.nn.sigmoid(pm_ref[...]) * SCALE
    s = jax.nn.softplus(ps_ref[...] + SIGMA_BOOST) + EPSILON
    eps = eps_ref[...]
    means_ref[...] = jnp.broadcast_to(m, eps.shape)
    sig_ref[...] = jnp.broadcast_to(s, eps.shape)
    samp = m + s * eps
    samp_ref[...] = samp
    idx_ref[...] = jnp.clip(jnp.round(samp), 0.0, SCALE).astype(jnp.int32)


_tc_call = pl.pallas_call(
    _tc_body,
    grid=(FLAT // N_BLK,),
    in_specs=[
        pl.BlockSpec((1, N_BLK), lambda i: (0, i)),
        pl.BlockSpec((1, N_BLK), lambda i: (0, i)),
        pl.BlockSpec((B, N_BLK), lambda i: (0, i)),
    ],
    out_specs=[
        pl.BlockSpec((B, N_BLK), lambda i: (0, i)),
        pl.BlockSpec((B, N_BLK), lambda i: (0, i)),
        pl.BlockSpec((B, N_BLK), lambda i: (0, i)),
        pl.BlockSpec((B, N_BLK), lambda i: (0, i)),
    ],
    out_shape=[
        jax.ShapeDtypeStruct((B, FLAT), jnp.float32),
        jax.ShapeDtypeStruct((B, FLAT), jnp.float32),
        jax.ShapeDtypeStruct((B, FLAT), jnp.float32),
        jax.ShapeDtypeStruct((B, FLAT), jnp.int32),
    ],
)


_NC = 2
_NS = 16
_ROWS_PER_W = B // (_NC * _NS)


@functools.cache
def _get_sc_contract():
    mesh = plsc.VectorSubcoreMesh(
        core_axis_name="c", subcore_axis_name="s",
        num_cores=_NC, num_subcores=_NS,
    )

    @functools.partial(
        pl.kernel,
        out_type=jax.ShapeDtypeStruct((B, N), jnp.float32),
        mesh=mesh,
        scratch_types=[
            pltpu.VMEM((N,), jnp.float32),
            pltpu.VMEM((FLAT + L,), jnp.int32),
            pltpu.VMEM((N,), jnp.float32),
            pltpu.VMEM((FLAT,), jnp.float32),
        ],
        compiler_params=pltpu.CompilerParams(needs_layout_passes=False),
    )
    def _sc_contract(x_hbm, idx_hbm, pv_hbm, y_hbm, xv, iv, yv, pv):
        wid = lax.axis_index("s") * _NC + lax.axis_index("c")
        pltpu.sync_copy(pv_hbm, pv)
        meven = (lax.iota(jnp.int32, L) % 2) == 0
        for rep in range(_ROWS_PER_W):
            b = wid * _ROWS_PER_W + rep
            pltpu.sync_copy(x_hbm.at[b], xv)
            pltpu.sync_copy(idx_hbm.at[b], iv.at[pl.ds(0, FLAT)])

            zero = jnp.zeros((L,), jnp.float32)

            def zbody(i, carry):
                yv[pl.ds(i * L, L)] = zero
                return carry

            lax.fori_loop(0, N // L, zbody, 0)

            def body(k, carry):
                base = k * L
                r = iv[pl.ds(base, L)]
                c = iv[pl.ds(base + 1, L)]
                g = plsc.load_gather(xv, [c], mask=meven)
                v = pv[pl.ds(base, L)]
                plsc.addupdate_scatter(yv, [r], g * v, mask=meven)
                return carry

            lax.fori_loop(0, FLAT // L, body, 0)
            pltpu.sync_copy(yv, y_hbm.at[b])

    return _sc_contract


def kernel(x, eps, pmeans, psigmas, pvalues):
    pm_flat = pmeans.reshape(1, FLAT)
    psig_rep = jnp.repeat(psigmas, 2).reshape(1, FLAT)
    eps_flat = eps.reshape(B, FLAT)
    means_f, sig_f, samp_f, idxi = _tc_call(pm_flat, psig_rep, eps_flat)
    pv_rep = jnp.repeat(pvalues, 2)
    y = _get_sc_contract()(x, idxi, pv_rep)
    return (
        y,
        means_f.reshape(B, N, 2),
        sig_f.reshape(B, N, 2),
        samp_f.reshape(B, N, 2),
    )


# speedup vs baseline: 22.2210x; 2.1439x over previous
import functools

import jax
import jax.numpy as jnp
from jax import lax
from jax.experimental import pallas as pl
from jax.experimental.pallas import tpu as pltpu
from jax.experimental.pallas import tpu_sc as plsc

N = 16384
B = 64
M = B * N
SCALE = float(N - 1)
SIGMA_BOOST = 2.0
EPSILON = 1e-7

L = 16


def _tc_body(pm_ref, ps_ref, eps_ref, means_ref, sig_ref, samp_ref, idx_ref):
    m = jax.nn.sigmoid(pm_ref[...]) * SCALE
    s = jax.nn.softplus(ps_ref[...] + SIGMA_BOOST) + EPSILON
    eps = eps_ref[...]
    means_ref[...] = m
    sig_ref[...] = jnp.broadcast_to(s, eps.shape)
    samp = m + s * eps
    samp_ref[...] = samp
    idx_ref[...] = jnp.clip(jnp.round(samp), 0.0, SCALE).astype(jnp.int32)


_tc_call = pl.pallas_call(
    _tc_body,
    grid=(B,),
    in_specs=[
        pl.BlockSpec((2, N), lambda i: (0, 0)),
        pl.BlockSpec((1, N), lambda i: (0, 0)),
        pl.BlockSpec((2, N), lambda i: (0, i)),
    ],
    out_specs=[
        pl.BlockSpec((2, N), lambda i: (0, i)),
        pl.BlockSpec((2, N), lambda i: (0, i)),
        pl.BlockSpec((2, N), lambda i: (0, i)),
        pl.BlockSpec((2, N), lambda i: (0, i)),
    ],
    out_shape=[
        jax.ShapeDtypeStruct((2, M), jnp.float32),
        jax.ShapeDtypeStruct((2, M), jnp.float32),
        jax.ShapeDtypeStruct((2, M), jnp.float32),
        jax.ShapeDtypeStruct((2, M), jnp.int32),
    ],
)


_NC = 2
_NS = 16
_ROWS_PER_W = B // (_NC * _NS)


@functools.cache
def _get_sc_contract():
    mesh = plsc.VectorSubcoreMesh(
        core_axis_name="c", subcore_axis_name="s",
        num_cores=_NC, num_subcores=_NS,
    )

    @functools.partial(
        pl.kernel,
        out_type=jax.ShapeDtypeStruct((B, N), jnp.float32),
        mesh=mesh,
        scratch_types=[
            pltpu.VMEM((N,), jnp.float32),
            pltpu.VMEM((2, N), jnp.int32),
            pltpu.VMEM((N,), jnp.float32),
            pltpu.VMEM((N,), jnp.float32),
        ],
        compiler_params=pltpu.CompilerParams(needs_layout_passes=False),
    )
    def _sc_contract(x_hbm, idx_hbm, pv_hbm, y_hbm, xv, iv, yv, pv):
        wid = lax.axis_index("s") * _NC + lax.axis_index("c")
        pltpu.sync_copy(pv_hbm, pv)
        for rep in range(_ROWS_PER_W):
            b = wid * _ROWS_PER_W + rep
            pltpu.sync_copy(x_hbm.at[b], xv)
            pltpu.sync_copy(idx_hbm.at[:, pl.ds(b * N, N)], iv)

            zero = jnp.zeros((L,), jnp.float32)

            def zbody(i, carry):
                yv[pl.ds(i * L, L)] = zero
                return carry

            lax.fori_loop(0, N // L, zbody, 0)

            def body(k, carry):
                base = k * L
                r = iv[0, pl.ds(base, L)]
                c = iv[1, pl.ds(base, L)]
                g = plsc.load_gather(xv, [c])
                v = pv[pl.ds(base, L)]
                plsc.addupdate_scatter(yv, [r], g * v)
                return carry

            lax.fori_loop(0, N // L, body, 0)
            pltpu.sync_copy(yv, y_hbm.at[b])

    return _sc_contract


def kernel(x, eps, pmeans, psigmas, pvalues):
    eps_pl = eps.reshape(M, 2).T
    pm_pl = pmeans.T
    ps2 = psigmas.reshape(1, N)
    means_pl, sig_pl, samp_pl, idx_pl = _tc_call(pm_pl, ps2, eps_pl)
    y = _get_sc_contract()(x, idx_pl, pvalues)
    return (
        y,
        means_pl.T.reshape(B, N, 2),
        sig_pl.T.reshape(B, N, 2),
        samp_pl.T.reshape(B, N, 2),
    )


# speedup vs baseline: 22.8726x; 1.0293x over previous
import functools

import jax
import jax.numpy as jnp
from jax import lax
from jax.experimental import pallas as pl
from jax.experimental.pallas import tpu as pltpu
from jax.experimental.pallas import tpu_sc as plsc

N = 16384
B = 64
M = B * N
SCALE = float(N - 1)
SIGMA_BOOST = 2.0
EPSILON = 1e-7

L = 16
UNROLL = 4


def _sample(pm, ps, eps):
    m = jax.nn.sigmoid(pm) * SCALE
    s = jax.nn.softplus(ps + SIGMA_BOOST) + EPSILON
    return m, s, m + s * eps


def _tc_idx_body(pm_ref, ps_ref, eps_ref, idx_ref):
    _, _, samp = _sample(pm_ref[...], ps_ref[...], eps_ref[...])
    idx_ref[...] = jnp.clip(jnp.round(samp), 0.0, SCALE).astype(jnp.int32)


def _tc_out_body(pm_ref, ps_ref, eps_ref, means_ref, sig_ref, samp_ref):
    m, s, samp = _sample(pm_ref[...], ps_ref[...], eps_ref[...])
    means_ref[...] = m
    sig_ref[...] = jnp.broadcast_to(s, samp.shape)
    samp_ref[...] = samp


_in_specs = [
    pl.BlockSpec((2, N), lambda i: (0, 0)),
    pl.BlockSpec((1, N), lambda i: (0, 0)),
    pl.BlockSpec((2, N), lambda i: (0, i)),
]

_tc_idx_call = pl.pallas_call(
    _tc_idx_body,
    grid=(B,),
    in_specs=_in_specs,
    out_specs=[pl.BlockSpec((2, N), lambda i: (0, i))],
    out_shape=[jax.ShapeDtypeStruct((2, M), jnp.int32)],
)

_tc_out_call = pl.pallas_call(
    _tc_out_body,
    grid=(B,),
    in_specs=_in_specs,
    out_specs=[pl.BlockSpec((2, N), lambda i: (0, i))] * 3,
    out_shape=[jax.ShapeDtypeStruct((2, M), jnp.float32)] * 3,
)


_NC = 2
_NS = 16
_ROWS_PER_W = B // (_NC * _NS)


@functools.cache
def _get_sc_contract():
    mesh = plsc.VectorSubcoreMesh(
        core_axis_name="c", subcore_axis_name="s",
        num_cores=_NC, num_subcores=_NS,
    )

    @functools.partial(
        pl.kernel,
        out_type=jax.ShapeDtypeStruct((B, N), jnp.float32),
        mesh=mesh,
        scratch_types=[
            pltpu.VMEM((N,), jnp.float32),
            pltpu.VMEM((2, N), jnp.int32),
            pltpu.VMEM((N,), jnp.float32),
            pltpu.VMEM((N,), jnp.float32),
        ],
        compiler_params=pltpu.CompilerParams(needs_layout_passes=False),
    )
    def _sc_contract(x_hbm, idx_hbm, pv_hbm, y_hbm, xv, iv, yv, pv):
        wid = lax.axis_index("s") * _NC + lax.axis_index("c")
        pltpu.sync_copy(pv_hbm, pv)
        for rep in range(_ROWS_PER_W):
            b = wid * _ROWS_PER_W + rep
            pltpu.sync_copy(x_hbm.at[b], xv)
            pltpu.sync_copy(idx_hbm.at[:, pl.ds(b * N, N)], iv)

            zero = jnp.zeros((L,), jnp.float32)

            def zbody(i, carry):
                for u in range(UNROLL):
                    yv[pl.ds((i * UNROLL + u) * L, L)] = zero
                return carry

            lax.fori_loop(0, N // L // UNROLL, zbody, 0)

            def body(k, carry):
                for u in range(UNROLL):
                    base = (k * UNROLL + u) * L
                    r = iv[0, pl.ds(base, L)]
                    c = iv[1, pl.ds(base, L)]
                    g = plsc.load_gather(xv, [c])
                    v = pv[pl.ds(base, L)]
                    plsc.addupdate_scatter(yv, [r], g * v)
                return carry

            lax.fori_loop(0, N // L // UNROLL, body, 0)
            pltpu.sync_copy(yv, y_hbm.at[b])

    return _sc_contract


def kernel(x, eps, pmeans, psigmas, pvalues):
    eps_pl = eps.reshape(M, 2).T
    pm_pl = pmeans.T
    ps2 = psigmas.reshape(1, N)
    (idx_pl,) = _tc_idx_call(pm_pl, ps2, eps_pl)
    y = _get_sc_contract()(x, idx_pl, pvalues)
    means_pl, sig_pl, samp_pl = _tc_out_call(pm_pl, ps2, eps_pl)
    return (
        y,
        means_pl.T.reshape(B, N, 2),
        sig_pl.T.reshape(B, N, 2),
        samp_pl.T.reshape(B, N, 2),
    )


# speedup vs baseline: 35.3047x; 1.5435x over previous
import functools

import jax
import jax.numpy as jnp
from jax import lax
from jax.experimental import pallas as pl
from jax.experimental.pallas import tpu as pltpu
from jax.experimental.pallas import tpu_sc as plsc

N = 16384
B = 64
M = B * N
SCALE = float(N - 1)
SIGMA_BOOST = 2.0
EPSILON = 1e-7

L = 16
UNROLL = 4


def _sample(pm, ps, eps):
    m = jax.nn.sigmoid(pm) * SCALE
    s = jax.nn.softplus(ps + SIGMA_BOOST) + EPSILON
    return m, s, m + s * eps


def _tc_idx_body(pm_ref, ps_ref, eps_ref, idx_ref):
    _, _, samp = _sample(pm_ref[...], ps_ref[...], eps_ref[...])
    idx_ref[...] = jnp.clip(jnp.round(samp), 0.0, SCALE).astype(jnp.int32)


def _tc_out_body(pm_ref, ps_ref, eps_ref, means_ref, sig_ref, samp_ref):
    m, s, samp = _sample(pm_ref[...], ps_ref[...], eps_ref[...])
    means_ref[...] = m
    sig_ref[...] = jnp.broadcast_to(s, samp.shape)
    samp_ref[...] = samp


TCW = 4
_E = TCW * N

_in_specs = [
    pl.BlockSpec((2, _E), lambda i: (0, 0)),
    pl.BlockSpec((1, _E), lambda i: (0, 0)),
    pl.BlockSpec((2, _E), lambda i: (0, i)),
]

_tc_idx_call = pl.pallas_call(
    _tc_idx_body,
    grid=(B // TCW,),
    in_specs=_in_specs,
    out_specs=[pl.BlockSpec((2, _E), lambda i: (0, i))],
    out_shape=[jax.ShapeDtypeStruct((2, M), jnp.int32)],
)

_tc_out_call = pl.pallas_call(
    _tc_out_body,
    grid=(B // TCW,),
    in_specs=_in_specs,
    out_specs=[pl.BlockSpec((2, _E), lambda i: (0, i))] * 3,
    out_shape=[jax.ShapeDtypeStruct((2, M), jnp.float32)] * 3,
)


_NC = 2
_NS = 16
_ROWS_PER_W = B // (_NC * _NS)


@functools.cache
def _get_sc_contract():
    mesh = plsc.VectorSubcoreMesh(
        core_axis_name="c", subcore_axis_name="s",
        num_cores=_NC, num_subcores=_NS,
    )

    @functools.partial(
        pl.kernel,
        out_type=jax.ShapeDtypeStruct((B, N), jnp.float32),
        mesh=mesh,
        scratch_types=[
            pltpu.VMEM((N,), jnp.float32),
            pltpu.VMEM((2, N), jnp.int32),
            pltpu.VMEM((N,), jnp.float32),
            pltpu.VMEM((N,), jnp.float32),
        ],
        compiler_params=pltpu.CompilerParams(needs_layout_passes=False),
    )
    def _sc_contract(x_hbm, idx_hbm, pv_hbm, y_hbm, xv, iv, yv, pv):
        wid = lax.axis_index("s") * _NC + lax.axis_index("c")
        pltpu.sync_copy(pv_hbm, pv)
        for rep in range(_ROWS_PER_W):
            b = wid * _ROWS_PER_W + rep
            pltpu.sync_copy(x_hbm.at[b], xv)
            pltpu.sync_copy(idx_hbm.at[:, pl.ds(b * N, N)], iv)

            zero = jnp.zeros((L,), jnp.float32)

            @plsc.parallel_loop(0, N // L, unroll=UNROLL)
            def _zbody(i):
                yv[pl.ds(i * L, L)] = zero

            @plsc.parallel_loop(0, N // L, unroll=UNROLL)
            def _body(k):
                base = k * L
                r = iv[0, pl.ds(base, L)]
                c = iv[1, pl.ds(base, L)]
                g = plsc.load_gather(xv, [c])
                v = pv[pl.ds(base, L)]
                plsc.addupdate_scatter(yv, [r], g * v)
            pltpu.sync_copy(yv, y_hbm.at[b])

    return _sc_contract


def kernel(x, eps, pmeans, psigmas, pvalues):
    eps_pl = eps.reshape(M, 2).T
    pm_pl = pmeans.T
    ps2 = psigmas.reshape(1, N)
    pm4 = jnp.tile(pm_pl, (1, TCW))
    ps4 = jnp.tile(ps2, (1, TCW))
    (idx_pl,) = _tc_idx_call(pm4, ps4, eps_pl)
    y = _get_sc_contract()(x, idx_pl, pvalues)
    means_pl, sig_pl, samp_pl = _tc_out_call(pm4, ps4, eps_pl)
    return (
        y,
        means_pl.T.reshape(B, N, 2),
        sig_pl.T.reshape(B, N, 2),
        samp_pl.T.reshape(B, N, 2),
    )


# speedup vs baseline: 36.3099x; 1.0285x over previous
import functools

import jax
import jax.numpy as jnp
from jax import lax
from jax.experimental import pallas as pl
from jax.experimental.pallas import tpu as pltpu
from jax.experimental.pallas import tpu_sc as plsc

N = 16384
B = 64
M = B * N
SCALE = float(N - 1)
SIGMA_BOOST = 2.0
EPSILON = 1e-7

L = 16
UNROLL = 8


def _sample(pm, ps, eps):
    m = jax.nn.sigmoid(pm) * SCALE
    s = jax.nn.softplus(ps + SIGMA_BOOST) + EPSILON
    return m, s, m + s * eps


def _tc_idx_body(pm_ref, ps_ref, eps_ref, idx_ref):
    _, _, samp = _sample(pm_ref[...], ps_ref[...], eps_ref[...])
    idx_ref[...] = jnp.clip(jnp.round(samp), 0.0, SCALE).astype(jnp.int32)


def _tc_out_body(pm_ref, ps_ref, eps_ref, means_ref, sig_ref, samp_ref):
    m, s, samp = _sample(pm_ref[...], ps_ref[...], eps_ref[...])
    means_ref[...] = m
    sig_ref[...] = jnp.broadcast_to(s, samp.shape)
    samp_ref[...] = samp


TCW = 8
_E = TCW * N

_in_specs = [
    pl.BlockSpec((2, _E), lambda i: (0, 0)),
    pl.BlockSpec((1, _E), lambda i: (0, 0)),
    pl.BlockSpec((2, _E), lambda i: (0, i)),
]

_tc_idx_call = pl.pallas_call(
    _tc_idx_body,
    grid=(B // TCW,),
    in_specs=_in_specs,
    out_specs=[pl.BlockSpec((2, _E), lambda i: (0, i))],
    out_shape=[jax.ShapeDtypeStruct((2, M), jnp.int32)],
)

_tc_out_call = pl.pallas_call(
    _tc_out_body,
    grid=(B // TCW,),
    in_specs=_in_specs,
    out_specs=[pl.BlockSpec((2, _E), lambda i: (0, i))] * 3,
    out_shape=[jax.ShapeDtypeStruct((2, M), jnp.float32)] * 3,
)


_NC = 2
_NS = 16
_ROWS_PER_W = B // (_NC * _NS)


@functools.cache
def _get_sc_contract():
    mesh = plsc.VectorSubcoreMesh(
        core_axis_name="c", subcore_axis_name="s",
        num_cores=_NC, num_subcores=_NS,
    )

    @functools.partial(
        pl.kernel,
        out_type=jax.ShapeDtypeStruct((B, N), jnp.float32),
        mesh=mesh,
        scratch_types=[
            pltpu.VMEM((N,), jnp.float32),
            pltpu.VMEM((2, N), jnp.int32),
            pltpu.VMEM((N,), jnp.float32),
            pltpu.VMEM((N,), jnp.float32),
        ],
        compiler_params=pltpu.CompilerParams(needs_layout_passes=False),
    )
    def _sc_contract(x_hbm, idx_hbm, pv_hbm, y_hbm, xv, iv, yv, pv):
        wid = lax.axis_index("s") * _NC + lax.axis_index("c")
        pltpu.sync_copy(pv_hbm, pv)
        for rep in range(_ROWS_PER_W):
            b = wid * _ROWS_PER_W + rep
            pltpu.sync_copy(x_hbm.at[b], xv)
            pltpu.sync_copy(idx_hbm.at[:, pl.ds(b * N, N)], iv)

            zero = jnp.zeros((L,), jnp.float32)

            @plsc.parallel_loop(0, N // L, unroll=UNROLL)
            def _zbody(i):
                yv[pl.ds(i * L, L)] = zero

            @plsc.parallel_loop(0, N // L, unroll=UNROLL)
            def _body(k):
                base = k * L
                r = iv[0, pl.ds(base, L)]
                c = iv[1, pl.ds(base, L)]
                g = plsc.load_gather(xv, [c])
                v = pv[pl.ds(base, L)]
                plsc.addupdate_scatter(yv, [r], g * v)
            pltpu.sync_copy(yv, y_hbm.at[b])

    return _sc_contract


def kernel(x, eps, pmeans, psigmas, pvalues):
    eps_pl = eps.reshape(M, 2).T
    pm_pl = pmeans.T
    ps2 = psigmas.reshape(1, N)
    pm4 = jnp.tile(pm_pl, (1, TCW))
    ps4 = jnp.tile(ps2, (1, TCW))
    (idx_pl,) = _tc_idx_call(pm4, ps4, eps_pl)
    y = _get_sc_contract()(x, idx_pl, pvalues)
    means_pl, sig_pl, samp_pl = _tc_out_call(pm4, ps4, eps_pl)
    return (
        y,
        means_pl.T.reshape(B, N, 2),
        sig_pl.T.reshape(B, N, 2),
        samp_pl.T.reshape(B, N, 2),
    )


# speedup vs baseline: 38.3674x; 1.0567x over previous
import functools

import jax
import jax.numpy as jnp
from jax import lax
from jax.experimental import pallas as pl
from jax.experimental.pallas import tpu as pltpu
from jax.experimental.pallas import tpu_sc as plsc

N = 16384
B = 64
M = B * N
SCALE = float(N - 1)
SIGMA_BOOST = 2.0
EPSILON = 1e-7

L = 16
UNROLL = 8


def _sample(pm, ps, eps):
    m = jax.nn.sigmoid(pm) * SCALE
    s = jax.nn.softplus(ps + SIGMA_BOOST) + EPSILON
    return m, s, m + s * eps


def _tc_idx_body(pm_ref, ps_ref, eps_ref, idx_ref):
    _, _, samp = _sample(pm_ref[...], ps_ref[...], eps_ref[...])
    idx_ref[...] = jnp.clip(jnp.round(samp), 0.0, SCALE).astype(jnp.int32)


def _tc_out_body(pm_ref, ps_ref, eps_ref, means_ref, sig_ref, samp_ref):
    m, s, samp = _sample(pm_ref[...], ps_ref[...], eps_ref[...])
    means_ref[...] = m
    sig_ref[...] = jnp.broadcast_to(s, samp.shape)
    samp_ref[...] = samp


TCW = 8
_E = TCW * N

_in_specs = [
    pl.BlockSpec((2, _E), lambda i: (0, 0)),
    pl.BlockSpec((1, _E), lambda i: (0, 0)),
    pl.BlockSpec((2, _E), lambda i: (0, i)),
]

_tc_idx_call = pl.pallas_call(
    _tc_idx_body,
    grid=(B // TCW,),
    in_specs=_in_specs,
    out_specs=[pl.BlockSpec((2, _E), lambda i: (0, i))],
    out_shape=[jax.ShapeDtypeStruct((2, M), jnp.int32)],
)

_tc_out_call = pl.pallas_call(
    _tc_out_body,
    grid=(B // TCW,),
    in_specs=_in_specs,
    out_specs=[pl.BlockSpec((2, _E), lambda i: (0, i))] * 3,
    out_shape=[jax.ShapeDtypeStruct((2, M), jnp.float32)] * 3,
)


_NC = 2
_NS = 16
_ROWS_PER_W = B // (_NC * _NS)


@functools.cache
def _get_sc_contract():
    mesh = plsc.VectorSubcoreMesh(
        core_axis_name="c", subcore_axis_name="s",
        num_cores=_NC, num_subcores=_NS,
    )

    @functools.partial(
        pl.kernel,
        out_type=jax.ShapeDtypeStruct((B, N), jnp.float32),
        mesh=mesh,
        scratch_types=[
            pltpu.VMEM((N,), jnp.float32),
            pltpu.VMEM((N,), jnp.float32),
            pltpu.VMEM((2, N), jnp.int32),
            pltpu.VMEM((N,), jnp.float32),
            pltpu.VMEM((N,), jnp.float32),
            pltpu.SemaphoreType.DMA,
            pltpu.SemaphoreType.DMA,
            pltpu.SemaphoreType.DMA,
            pltpu.SemaphoreType.DMA,
            pltpu.SemaphoreType.DMA,
        ],
        compiler_params=pltpu.CompilerParams(needs_layout_passes=False),
    )
    def _sc_contract(x_hbm, idx_hbm, pv_hbm, y_hbm, xv0, xv1, iv, yv, pv,
                     sempv, semx0, semx1, semi, semy):
        wid = lax.axis_index("s") * _NC + lax.axis_index("c")
        b0 = wid * _ROWS_PER_W
        zero = jnp.zeros((L,), jnp.float32)

        cp_pv = pltpu.async_copy(pv_hbm, pv, sempv)
        cp_x0 = pltpu.async_copy(x_hbm.at[b0], xv0, semx0)
        cp_i0 = pltpu.async_copy(idx_hbm.at[:, pl.ds(b0 * N, N)], iv, semi)
        cp_x1 = pltpu.async_copy(x_hbm.at[b0 + 1], xv1, semx1)

        @plsc.parallel_loop(0, N // L, unroll=UNROLL)
        def _z0(i):
            yv[pl.ds(i * L, L)] = zero

        cp_pv.wait()
        cp_x0.wait()
        cp_i0.wait()

        @plsc.parallel_loop(0, N // L, unroll=UNROLL)
        def _b1(k):
            base = k * L
            r = iv[0, pl.ds(base, L)]
            c = iv[1, pl.ds(base, L)]
            g = plsc.load_gather(xv0, [c])
            v = pv[pl.ds(base, L)]
            plsc.addupdate_scatter(yv, [r], g * v)

        cp_y0 = pltpu.async_copy(yv, y_hbm.at[b0], semy)
        cp_i1 = pltpu.async_copy(idx_hbm.at[:, pl.ds((b0 + 1) * N, N)], iv, semi)
        cp_x1.wait()

        y2 = xv0

        @plsc.parallel_loop(0, N // L, unroll=UNROLL)
        def _z1(i):
            y2[pl.ds(i * L, L)] = zero

        cp_i1.wait()

        @plsc.parallel_loop(0, N // L, unroll=UNROLL)
        def _b2(k):
            base = k * L
            r = iv[0, pl.ds(base, L)]
            c = iv[1, pl.ds(base, L)]
            g = plsc.load_gather(xv1, [c])
            v = pv[pl.ds(base, L)]
            plsc.addupdate_scatter(y2, [r], g * v)

        cp_y0.wait()
        pltpu.sync_copy(y2, y_hbm.at[b0 + 1])

    return _sc_contract


def kernel(x, eps, pmeans, psigmas, pvalues):
    eps_pl = eps.reshape(M, 2).T
    pm_pl = pmeans.T
    ps2 = psigmas.reshape(1, N)
    pm4 = jnp.tile(pm_pl, (1, TCW))
    ps4 = jnp.tile(ps2, (1, TCW))
    (idx_pl,) = _tc_idx_call(pm4, ps4, eps_pl)
    y = _get_sc_contract()(x, idx_pl, pvalues)
    means_pl, sig_pl, samp_pl = _tc_out_call(pm4, ps4, eps_pl)
    return (
        y,
        means_pl.T.reshape(B, N, 2),
        sig_pl.T.reshape(B, N, 2),
        samp_pl.T.reshape(B, N, 2),
    )


# speedup vs baseline: 41.4229x; 1.0796x over previous
import functools

import jax
import jax.numpy as jnp
from jax import lax
from jax.experimental import pallas as pl
from jax.experimental.pallas import tpu as pltpu
from jax.experimental.pallas import tpu_sc as plsc

N = 16384
B = 64
M = B * N
SCALE = float(N - 1)
SIGMA_BOOST = 2.0
EPSILON = 1e-7

L = 16
UNROLL = 8
TCW = 8
_E = TCW * N


def _sample(pm, ps, eps):
    m1 = jax.nn.sigmoid(pm) * SCALE
    s1 = jax.nn.softplus(ps + SIGMA_BOOST) + EPSILON
    m = jnp.concatenate([m1] * TCW, axis=1)
    s = jnp.concatenate([s1] * TCW, axis=1)
    return m, s, m + s * eps


def _tc_idx_body(pm_ref, ps_ref, eps_ref, idx_ref):
    _, _, samp = _sample(pm_ref[...], ps_ref[...], eps_ref[...])
    idx_ref[...] = jnp.clip(jnp.round(samp), 0.0, SCALE).astype(jnp.int32)


def _tc_out_body(pm_ref, ps_ref, eps_ref, means_ref, sig_ref, samp_ref):
    m, s, samp = _sample(pm_ref[...], ps_ref[...], eps_ref[...])
    means_ref[...] = m
    sig_ref[...] = jnp.broadcast_to(s, samp.shape)
    samp_ref[...] = samp


_in_specs = [
    pl.BlockSpec((2, N), lambda i: (0, 0)),
    pl.BlockSpec((1, N), lambda i: (0, 0)),
    pl.BlockSpec((2, _E), lambda i: (0, i)),
]

_tc_idx_call = pl.pallas_call(
    _tc_idx_body,
    grid=(B // TCW,),
    in_specs=_in_specs,
    out_specs=[pl.BlockSpec((2, _E), lambda i: (0, i))],
    out_shape=[jax.ShapeDtypeStruct((2, M), jnp.int32)],
)

_tc_out_call = pl.pallas_call(
    _tc_out_body,
    grid=(B // TCW,),
    in_specs=_in_specs,
    out_specs=[pl.BlockSpec((2, _E), lambda i: (0, i))] * 3,
    out_shape=[jax.ShapeDtypeStruct((2, M), jnp.float32)] * 3,
)


_NC = 2
_NS = 16
_ROWS_PER_W = B // (_NC * _NS)


@functools.cache
def _get_sc_contract():
    mesh = plsc.VectorSubcoreMesh(
        core_axis_name="c", subcore_axis_name="s",
        num_cores=_NC, num_subcores=_NS,
    )

    @functools.partial(
        pl.kernel,
        out_type=jax.ShapeDtypeStruct((B, N), jnp.float32),
        mesh=mesh,
        scratch_types=[
            pltpu.VMEM((N,), jnp.float32),
            pltpu.VMEM((N,), jnp.float32),
            pltpu.VMEM((2, N), jnp.int32),
            pltpu.VMEM((N,), jnp.float32),
            pltpu.VMEM((N,), jnp.float32),
            pltpu.SemaphoreType.DMA,
            pltpu.SemaphoreType.DMA,
            pltpu.SemaphoreType.DMA,
            pltpu.SemaphoreType.DMA,
            pltpu.SemaphoreType.DMA,
        ],
        compiler_params=pltpu.CompilerParams(needs_layout_passes=False),
    )
    def _sc_contract(x_hbm, idx_hbm, pv_hbm, y_hbm, xv0, xv1, iv, yv, pv,
                     sempv, semx0, semx1, semi, semy):
        wid = lax.axis_index("s") * _NC + lax.axis_index("c")
        b0 = wid * _ROWS_PER_W
        zero = jnp.zeros((L,), jnp.float32)

        cp_pv = pltpu.async_copy(pv_hbm, pv, sempv)
        cp_x0 = pltpu.async_copy(x_hbm.at[b0], xv0, semx0)
        cp_i0 = pltpu.async_copy(idx_hbm.at[:, pl.ds(b0 * N, N)], iv, semi)
        cp_x1 = pltpu.async_copy(x_hbm.at[b0 + 1], xv1, semx1)

        @plsc.parallel_loop(0, N // L, unroll=UNROLL)
        def _z0(i):
            yv[pl.ds(i * L, L)] = zero

        cp_pv.wait()
        cp_x0.wait()
        cp_i0.wait()

        @plsc.parallel_loop(0, N // L, unroll=UNROLL)
        def _b1(k):
            base = k * L
            r = iv[0, pl.ds(base, L)]
            c = iv[1, pl.ds(base, L)]
            g = plsc.load_gather(xv0, [c])
            v = pv[pl.ds(base, L)]
            plsc.addupdate_scatter(yv, [r], g * v)

        cp_y0 = pltpu.async_copy(yv, y_hbm.at[b0], semy)
        cp_i1 = pltpu.async_copy(idx_hbm.at[:, pl.ds((b0 + 1) * N, N)], iv, semi)
        cp_x1.wait()

        y2 = xv0

        @plsc.parallel_loop(0, N // L, unroll=UNROLL)
        def _z1(i):
            y2[pl.ds(i * L, L)] = zero

        cp_i1.wait()

        @plsc.parallel_loop(0, N // L, unroll=UNROLL)
        def _b2(k):
            base = k * L
            r = iv[0, pl.ds(base, L)]
            c = iv[1, pl.ds(base, L)]
            g = plsc.load_gather(xv1, [c])
            v = pv[pl.ds(base, L)]
            plsc.addupdate_scatter(y2, [r], g * v)

        cp_y0.wait()
        pltpu.sync_copy(y2, y_hbm.at[b0 + 1])

    return _sc_contract


def kernel(x, eps, pmeans, psigmas, pvalues):
    eps_pl = eps.reshape(M, 2).T
    pm_pl = pmeans.T
    ps2 = psigmas.reshape(1, N)
    (idx_pl,) = _tc_idx_call(pm_pl, ps2, eps_pl)
    y = _get_sc_contract()(x, idx_pl, pvalues)
    means_pl, sig_pl, samp_pl = _tc_out_call(pm_pl, ps2, eps_pl)
    return (
        y,
        means_pl.T.reshape(B, N, 2),
        sig_pl.T.reshape(B, N, 2),
        samp_pl.T.reshape(B, N, 2),
    )
